# baseline (speedup 1.0000x reference)
import functools

import jax
import jax.numpy as jnp
from jax import lax
from jax.experimental import pallas as pl
from jax.experimental.pallas import tpu as pltpu
from jax.experimental.pallas import tpu_sc as plsc

B = 128
V = 100000
L = 16
NG = B // L
NC, NS = 2, 16
NW = NC * NS
RPW = 4
VR = 344
NCH = 9
TAILR = 32
NEG = -3.0e38
LN2 = 0.6931471805599453


def _allreduce16(x, t, op):
    for k in (8, 4, 2, 1):
        t[pl.ds(0, L)] = x
        t[pl.ds(L, L)] = x
        x = op(x, t[pl.ds(k, L)])
    return x


def _log16(x):
    b = lax.bitcast_convert_type(x, jnp.int32)
    e = ((b >> 23) & 0xFF) - 127
    f = lax.bitcast_convert_type((b & 0x7FFFFF) | 0x3F800000, jnp.float32)
    u = f - 1.0
    y = e.astype(jnp.float32) * LN2 + u * (1.0 - 0.5 * u)
    for _ in range(3):
        y = y + x * jnp.exp(-y) - 1.0
    return y


def _reduce_body(lt_hbm, part_hbm, buf0, buf1, tbuf, st_v, red_unused,
                 sem0, sem1, semt):
    wid = lax.axis_index("s") * NC + lax.axis_index("c")
    v0 = wid * 3120 + 8 * jnp.minimum(wid, 20)
    vlen = 3120 + 8 * jnp.where(wid < 20, 1, 0)

    for j in range(NG):
        st_v[pl.ds(j * L, L)] = jnp.full((L,), NEG, jnp.float32)
        st_v[pl.ds((NG + j) * L, L)] = jnp.zeros((L,), jnp.float32)

    def issue(row, buf, sem, nrows):
        pltpu.make_async_copy(
            lt_hbm.at[pl.ds(row, nrows), :], buf, sem).start()

    def wait(buf, sem, nrows):
        pltpu.make_async_copy(
            lt_hbm.at[pl.ds(0, nrows), :], buf, sem).wait()

    def process(buf, lo, hi):
        m = [st_v[pl.ds(j * L, L)] for j in range(NG)]
        s = [st_v[pl.ds((NG + j) * L, L)] for j in range(NG)]

        @plsc.parallel_loop(lo, hi, step=16, carry=tuple(m))
        def new_m(i, acc):
            acc = list(acc)
            rb = i // NG
            for g in range(NG):
                x0 = buf[rb, pl.ds(g * L, L)]
                x1 = buf[rb + 1, pl.ds(g * L, L)]
                acc[g] = jnp.maximum(acc[g], jnp.maximum(x0, x1))
            return tuple(acc)

        s = [s[j] * jnp.exp(m[j] - new_m[j]) for j in range(NG)]

        @plsc.parallel_loop(lo, hi, step=16, carry=tuple(s))
        def new_s(i, acc):
            acc = list(acc)
            rb = i // NG
            for g in range(NG):
                e = [jnp.exp(buf[rb + r, pl.ds(g * L, L)] - new_m[g])
                     for r in range(2)]
                acc[g] = acc[g] + (e[0] + e[1])
            return tuple(acc)

        for j in range(NG):
            st_v[pl.ds(j * L, L)] = new_m[j]
            st_v[pl.ds((NG + j) * L, L)] = new_s[j]

    tstart = jnp.minimum(v0 + NCH * VR, V - TAILR)
    tlo = (v0 + NCH * VR - tstart) * NG
    thi = tlo + (vlen - NCH * VR) * NG

    issue(v0, buf0, sem0, VR)
    issue(v0 + VR, buf1, sem1, VR)
    issue(tstart, tbuf, semt, TAILR)

    def pair(i, _):
        wait(buf0, sem0, VR)
        process(buf0, 0, VR * NG)

        @pl.when(i < 4)
        def _i0():
            issue(v0 + (2 * i + 2) * VR, buf0, sem0, VR)

        wait(buf1, sem1, VR)
        process(buf1, 0, VR * NG)

        @pl.when(i < 3)
        def _i1():
            issue(v0 + (2 * i + 3) * VR, buf1, sem1, VR)

        return _

    lax.fori_loop(0, 4, pair, None)
    wait(buf0, sem0, VR)
    process(buf0, 0, VR * NG)
    wait(tbuf, semt, TAILR)
    process(tbuf, tlo, thi)

    pltpu.sync_copy(st_v, part_hbm.at[wid])


def _combine_body(lt_hbm, value_hbm, part_hbm, out_hbm,
                  part_v, val_v, gbuf, out_v, red_t, val_sh, val_s, semg):
    wid = lax.axis_index("s") * NC + lax.axis_index("c")
    g0 = wid // 4
    iota = lax.iota(jnp.int32, L)

    pltpu.sync_copy(value_hbm, val_v)
    pltpu.sync_copy(val_v, val_sh)
    pltpu.sync_copy(val_sh, val_s)
    pltpu.sync_copy(part_hbm, part_v)

    vrow = []
    for rr in range(RPW):
        v = val_s[RPW * wid + rr]
        a8 = (v // 8) * 8
        pltpu.make_async_copy(
            lt_hbm.at[pl.ds(a8, 8), :], gbuf.at[pl.ds(rr * 8, 8), :],
            semg).start()
        vrow.append((v, a8))

    mt = [part_v[t, pl.ds(g0 * L, L)] for t in range(NW)]
    m_tot = mt[0]
    for t in range(1, NW):
        m_tot = jnp.maximum(m_tot, mt[t])
    s_tot = jnp.zeros((L,), jnp.float32)
    for t in range(NW):
        st = part_v[t, pl.ds((NG + g0) * L, L)]
        s_tot = s_tot + st * jnp.exp(mt[t] - m_tot)

    for rr in range(RPW):
        pltpu.make_async_copy(
            lt_hbm.at[pl.ds(0, 8), :], gbuf.at[pl.ds(rr * 8, 8), :],
            semg).wait()

    gv = jnp.zeros((L,), jnp.float32)
    sv = jnp.ones((L,), jnp.float32)
    vlog = jnp.zeros((L,), jnp.float32)
    for rr in range(RPW):
        lane = (RPW * wid + rr) % L
        v, a8 = vrow[rr]
        xg = gbuf[rr * 8 + (v - a8), pl.ds(g0 * L, L)]
        vl = _allreduce16(jnp.where(iota == lane, xg, 0.0), red_t, jnp.add)
        mr = _allreduce16(jnp.where(iota == lane, m_tot, 0.0), red_t, jnp.add)
        sr = _allreduce16(jnp.where(iota == lane, s_tot, 0.0), red_t, jnp.add)
        gv = jnp.where(iota == rr, mr, gv)
        sv = jnp.where(iota == rr, sr, sv)
        vlog = jnp.where(iota == rr, vl, vlog)
    out_v[...] = vlog - gv - _log16(sv)
    pltpu.sync_copy(out_v, out_hbm.at[wid])


@jax.jit
def kernel(logits, value):
    mesh = plsc.VectorSubcoreMesh(
        core_axis_name="c", subcore_axis_name="s",
        num_cores=NC, num_subcores=NS)
    lt = logits.T
    reduce_run = functools.partial(
        pl.kernel,
        out_type=jax.ShapeDtypeStruct((NW, 2 * NG * L), jnp.float32),
        mesh=mesh,
        scratch_types=[
            pltpu.VMEM((VR, B), jnp.float32),
            pltpu.VMEM((VR, B), jnp.float32),
            pltpu.VMEM((TAILR, B), jnp.float32),
            pltpu.VMEM((2 * NG * L,), jnp.float32),
            pltpu.VMEM((2 * L,), jnp.float32),
            pltpu.SemaphoreType.DMA,
            pltpu.SemaphoreType.DMA,
            pltpu.SemaphoreType.DMA,
        ],
    )(_reduce_body)
    combine_run = functools.partial(
        pl.kernel,
        out_type=jax.ShapeDtypeStruct((NW, L), jnp.float32),
        mesh=mesh,
        scratch_types=[
            pltpu.VMEM((NW, 2 * NG * L), jnp.float32),
            pltpu.VMEM((B,), jnp.int32),
            pltpu.VMEM((RPW * 8, B), jnp.float32),
            pltpu.VMEM((L,), jnp.float32),
            pltpu.VMEM((2 * L,), jnp.float32),
            pltpu.VMEM_SHARED((B,), jnp.int32),
            pltpu.SMEM((B,), jnp.int32),
            pltpu.SemaphoreType.DMA,
        ],
    )(_combine_body)
    vi = value.astype(jnp.int32)
    part = reduce_run(lt)
    out2d = combine_run(lt, vi, part)
    return out2d[:, :RPW].reshape(B)


# speedup vs baseline: 1.0025x; 1.0025x over previous
import functools

import jax
import jax.numpy as jnp
from jax import lax
from jax.experimental import pallas as pl
from jax.experimental.pallas import tpu as pltpu
from jax.experimental.pallas import tpu_sc as plsc

B = 128
V = 100000
L = 16
NG = B // L
NC, NS = 2, 16
NW = NC * NS
RPW = 4
VR = 344
NCH = 9
TAILR = 32
NEG = -3.0e38
LN2 = 0.6931471805599453


def _allreduce16(x, t, op):
    for k in (8, 4, 2, 1):
        t[pl.ds(0, L)] = x
        t[pl.ds(L, L)] = x
        x = op(x, t[pl.ds(k, L)])
    return x


def _log16(x):
    b = lax.bitcast_convert_type(x, jnp.int32)
    e = ((b >> 23) & 0xFF) - 127
    f = lax.bitcast_convert_type((b & 0x7FFFFF) | 0x3F800000, jnp.float32)
    u = f - 1.0
    y = e.astype(jnp.float32) * LN2 + u * (1.0 - 0.5 * u)
    for _ in range(3):
        y = y + x * jnp.exp(-y) - 1.0
    return y


def _reduce_body(lt_hbm, part_hbm, buf0, buf1, tbuf, st_v, red_unused,
                 sem0, sem1, semt):
    wid = lax.axis_index("s") * NC + lax.axis_index("c")
    v0 = wid * 3120 + 8 * jnp.minimum(wid, 20)
    vlen = 3120 + 8 * jnp.where(wid < 20, 1, 0)

    for j in range(NG):
        st_v[pl.ds(j * L, L)] = jnp.full((L,), NEG, jnp.float32)
        st_v[pl.ds((NG + j) * L, L)] = jnp.zeros((L,), jnp.float32)

    def issue(row, buf, sem, nrows):
        h = (nrows // 16) * 8
        pltpu.make_async_copy(
            lt_hbm.at[pl.ds(row, h), :], buf.at[pl.ds(0, h), :], sem).start()
        pltpu.make_async_copy(
            lt_hbm.at[pl.ds(row + h, nrows - h), :],
            buf.at[pl.ds(h, nrows - h), :], sem).start()

    def wait(buf, sem, nrows):
        h = (nrows // 16) * 8
        pltpu.make_async_copy(
            lt_hbm.at[pl.ds(0, h), :], buf.at[pl.ds(0, h), :], sem).wait()
        pltpu.make_async_copy(
            lt_hbm.at[pl.ds(0, nrows - h), :],
            buf.at[pl.ds(h, nrows - h), :], sem).wait()

    def process(buf, lo, hi):
        m = [st_v[pl.ds(j * L, L)] for j in range(NG)]
        s = [st_v[pl.ds((NG + j) * L, L)] for j in range(NG)]

        @plsc.parallel_loop(lo, hi, step=16, carry=tuple(m))
        def new_m(i, acc):
            acc = list(acc)
            rb = i // NG
            for g in range(NG):
                x0 = buf[rb, pl.ds(g * L, L)]
                x1 = buf[rb + 1, pl.ds(g * L, L)]
                acc[g] = jnp.maximum(acc[g], jnp.maximum(x0, x1))
            return tuple(acc)

        s = [s[j] * jnp.exp(m[j] - new_m[j]) for j in range(NG)]

        @plsc.parallel_loop(lo, hi, step=16, carry=tuple(s))
        def new_s(i, acc):
            acc = list(acc)
            rb = i // NG
            for g in range(NG):
                e = [jnp.exp(buf[rb + r, pl.ds(g * L, L)] - new_m[g])
                     for r in range(2)]
                acc[g] = acc[g] + (e[0] + e[1])
            return tuple(acc)

        for j in range(NG):
            st_v[pl.ds(j * L, L)] = new_m[j]
            st_v[pl.ds((NG + j) * L, L)] = new_s[j]

    tstart = jnp.minimum(v0 + NCH * VR, V - TAILR)
    tlo = (v0 + NCH * VR - tstart) * NG
    thi = tlo + (vlen - NCH * VR) * NG

    issue(v0, buf0, sem0, VR)
    issue(v0 + VR, buf1, sem1, VR)
    issue(tstart, tbuf, semt, TAILR)

    def pair(i, _):
        wait(buf0, sem0, VR)
        process(buf0, 0, VR * NG)

        @pl.when(i < 4)
        def _i0():
            issue(v0 + (2 * i + 2) * VR, buf0, sem0, VR)

        wait(buf1, sem1, VR)
        process(buf1, 0, VR * NG)

        @pl.when(i < 3)
        def _i1():
            issue(v0 + (2 * i + 3) * VR, buf1, sem1, VR)

        return _

    lax.fori_loop(0, 4, pair, None)
    wait(buf0, sem0, VR)
    process(buf0, 0, VR * NG)
    wait(tbuf, semt, TAILR)
    process(tbuf, tlo, thi)

    pltpu.sync_copy(st_v, part_hbm.at[wid])


def _combine_body(lt_hbm, value_hbm, part_hbm, out_hbm,
                  part_v, val_v, gbuf, out_v, red_t, val_sh, val_s, semg):
    wid = lax.axis_index("s") * NC + lax.axis_index("c")
    g0 = wid // 4
    iota = lax.iota(jnp.int32, L)

    pltpu.sync_copy(value_hbm, val_v)
    pltpu.sync_copy(val_v, val_sh)
    pltpu.sync_copy(val_sh, val_s)
    pltpu.sync_copy(part_hbm, part_v)

    vrow = []
    for rr in range(RPW):
        v = val_s[RPW * wid + rr]
        a8 = (v // 8) * 8
        pltpu.make_async_copy(
            lt_hbm.at[pl.ds(a8, 8), :], gbuf.at[pl.ds(rr * 8, 8), :],
            semg).start()
        vrow.append((v, a8))

    mt = [part_v[t, pl.ds(g0 * L, L)] for t in range(NW)]
    m_tot = mt[0]
    for t in range(1, NW):
        m_tot = jnp.maximum(m_tot, mt[t])
    s_tot = jnp.zeros((L,), jnp.float32)
    for t in range(NW):
        st = part_v[t, pl.ds((NG + g0) * L, L)]
        s_tot = s_tot + st * jnp.exp(mt[t] - m_tot)

    for rr in range(RPW):
        pltpu.make_async_copy(
            lt_hbm.at[pl.ds(0, 8), :], gbuf.at[pl.ds(rr * 8, 8), :],
            semg).wait()

    gv = jnp.zeros((L,), jnp.float32)
    sv = jnp.ones((L,), jnp.float32)
    vlog = jnp.zeros((L,), jnp.float32)
    for rr in range(RPW):
        lane = (RPW * wid + rr) % L
        v, a8 = vrow[rr]
        xg = gbuf[rr * 8 + (v - a8), pl.ds(g0 * L, L)]
        vl = _allreduce16(jnp.where(iota == lane, xg, 0.0), red_t, jnp.add)
        mr = _allreduce16(jnp.where(iota == lane, m_tot, 0.0), red_t, jnp.add)
        sr = _allreduce16(jnp.where(iota == lane, s_tot, 0.0), red_t, jnp.add)
        gv = jnp.where(iota == rr, mr, gv)
        sv = jnp.where(iota == rr, sr, sv)
        vlog = jnp.where(iota == rr, vl, vlog)
    out_v[...] = vlog - gv - _log16(sv)
    pltpu.sync_copy(out_v, out_hbm.at[wid])


@jax.jit
def kernel(logits, value):
    mesh = plsc.VectorSubcoreMesh(
        core_axis_name="c", subcore_axis_name="s",
        num_cores=NC, num_subcores=NS)
    lt = logits.T
    reduce_run = functools.partial(
        pl.kernel,
        out_type=jax.ShapeDtypeStruct((NW, 2 * NG * L), jnp.float32),
        mesh=mesh,
        scratch_types=[
            pltpu.VMEM((VR, B), jnp.float32),
            pltpu.VMEM((VR, B), jnp.float32),
            pltpu.VMEM((TAILR, B), jnp.float32),
            pltpu.VMEM((2 * NG * L,), jnp.float32),
            pltpu.VMEM((2 * L,), jnp.float32),
            pltpu.SemaphoreType.DMA,
            pltpu.SemaphoreType.DMA,
            pltpu.SemaphoreType.DMA,
        ],
    )(_reduce_body)
    combine_run = functools.partial(
        pl.kernel,
        out_type=jax.ShapeDtypeStruct((NW, L), jnp.float32),
        mesh=mesh,
        scratch_types=[
            pltpu.VMEM((NW, 2 * NG * L), jnp.float32),
            pltpu.VMEM((B,), jnp.int32),
            pltpu.VMEM((RPW * 8, B), jnp.float32),
            pltpu.VMEM((L,), jnp.float32),
            pltpu.VMEM((2 * L,), jnp.float32),
            pltpu.VMEM_SHARED((B,), jnp.int32),
            pltpu.SMEM((B,), jnp.int32),
            pltpu.SemaphoreType.DMA,
        ],
    )(_combine_body)
    vi = value.astype(jnp.int32)
    part = reduce_run(lt)
    out2d = combine_run(lt, vi, part)
    return out2d[:, :RPW].reshape(B)
